# baseline (speedup 1.0000x reference)
import jax
import jax.numpy as jnp
from jax.experimental import pallas as pl
from jax.experimental.pallas import tpu as pltpu


def _add_row_kernel(idx_ref, x_ref, row_ref, o_ref):
    o_ref[...] = x_ref[...] + row_ref[0]


def kernel(rep_idx, x, emb):
    B, S, D = x.shape
    N = B * S
    xf = x.reshape(N, D)
    BLK = 1024
    idx = jnp.asarray(rep_idx, jnp.int32).reshape(1)
    grid_spec = pltpu.PrefetchScalarGridSpec(
        num_scalar_prefetch=1,
        grid=(N // BLK,),
        in_specs=[
            pl.BlockSpec((BLK, D), lambda i, idx_ref: (i, 0)),
            pl.BlockSpec((1, 1, D), lambda i, idx_ref: (idx_ref[0], 0, 0)),
        ],
        out_specs=pl.BlockSpec((BLK, D), lambda i, idx_ref: (i, 0)),
    )
    out = pl.pallas_call(
        _add_row_kernel,
        grid_spec=grid_spec,
        out_shape=jax.ShapeDtypeStruct((N, D), x.dtype),
    )(idx, xf, emb.reshape(emb.shape[0], 1, D))
    return out.reshape(B, S, D)
